# Initial kernel scaffold; baseline (speedup 1.0000x reference)
import jax
import jax.numpy as jnp
from jax.experimental import pallas as pl


def kernel(z, pos, batch, edge_index, params):
    raise NotImplementedError("write your pallas kernel here")


# speedup vs baseline: 2.7093x; 2.7093x over previous
import functools

import jax
import jax.numpy as jnp
from jax import lax
from jax.experimental import pallas as pl
from jax.experimental.pallas import tpu as pltpu
from jax.experimental.pallas import tpu_sc as plsc

N = 10000
E = 320000
G = 256
F = 128
D = 100
CUT = 12.0

NC = 2
NS = 16
L = 16
NW = NC * NS

K = 128
E_PAD = 323584
EPW = E_PAD // NW
CHUNKS = EPW // K
N_PAD = 10240
NSLICE = N_PAD // NS
BE = 512
EROWS = E_PAD // BE
NB = N_PAD // 256

@functools.cache
def _get_mesh():
    return plsc.VectorSubcoreMesh(core_axis_name="c", subcore_axis_name="s",
                                  num_cores=NC, num_subcores=NS)


def _sc_d2_body(row_hbm, col_hbm, px_hbm, py_hbm, pz_hbm, d2_hbm,
                row_v, col_v, px_v, py_v, pz_v, d2_v):
    cid = lax.axis_index("c")
    sid = lax.axis_index("s")
    wid = sid * NC + cid
    base = wid * EPW
    pltpu.sync_copy(row_hbm.at[pl.ds(base, EPW)], row_v)
    pltpu.sync_copy(col_hbm.at[pl.ds(base, EPW)], col_v)
    pltpu.sync_copy(px_hbm, px_v)
    pltpu.sync_copy(py_hbm, py_v)
    pltpu.sync_copy(pz_hbm, pz_v)

    def step(i, carry):
        sl = pl.ds(i * L, L)
        r = row_v[sl]
        c = col_v[sl]
        dx = plsc.load_gather(px_v, [r]) - plsc.load_gather(px_v, [c])
        dy = plsc.load_gather(py_v, [r]) - plsc.load_gather(py_v, [c])
        dz = plsc.load_gather(pz_v, [r]) - plsc.load_gather(pz_v, [c])
        d2_v[sl] = dx * dx + dy * dy + dz * dz
        return carry

    lax.fori_loop(0, EPW // L, step, 0)
    pltpu.sync_copy(d2_v, d2_hbm.at[pl.ds(base, EPW)])


@jax.jit
def _sc_d2(row, col, px, py, pz):
    return pl.kernel(
        _sc_d2_body,
        out_type=jax.ShapeDtypeStruct((E_PAD,), jnp.float32),
        mesh=_get_mesh(),
        compiler_params=pltpu.CompilerParams(needs_layout_passes=False),
        scratch_types=[
            pltpu.VMEM((EPW,), jnp.int32),
            pltpu.VMEM((EPW,), jnp.int32),
            pltpu.VMEM((N_PAD,), jnp.float32),
            pltpu.VMEM((N_PAD,), jnp.float32),
            pltpu.VMEM((N_PAD,), jnp.float32),
            pltpu.VMEM((EPW,), jnp.float32),
        ],
    )(row, col, px, py, pz)


def _sc_msg_body(y_hbm, g_hbm, row_hbm, col_hbm, zero_hbm, out_hbm,
                 row_v, col_v, y_v, g_v, sem, m_sh):
    cid = lax.axis_index("c")
    sid = lax.axis_index("s")
    wid = sid * NC + cid
    base = wid * EPW
    zsl = pl.ds(sid * NSLICE, NSLICE)
    pltpu.sync_copy(zero_hbm.at[zsl], m_sh.at[zsl])
    plsc.subcore_barrier()

    def chunk(k, carry):
        ebase = base + k * K
        pltpu.sync_copy(row_hbm.at[pl.ds(ebase, K)], row_v)
        pltpu.sync_copy(col_hbm.at[pl.ds(ebase, K)], col_v)
        pltpu.async_copy(y_hbm.at[row_v], y_v, sem).wait()
        pltpu.sync_copy(g_hbm.at[pl.ds(ebase, K)], g_v)

        def mul_row(r, c2):
            for kk in range(F // L):
                sl = pl.ds(kk * L, L)
                g_v[r, sl] = g_v[r, sl] * y_v[r, sl]
            return c2

        lax.fori_loop(0, K, mul_row, 0)
        pltpu.sync_copy(g_v, m_sh.at[col_v], add=True)
        return carry

    lax.fori_loop(0, CHUNKS, chunk, 0)
    plsc.subcore_barrier()
    pltpu.sync_copy(m_sh.at[zsl], out_hbm.at[cid, zsl])


@jax.jit
def _sc_msg(y, g, row, col, zero):
    return pl.kernel(
        _sc_msg_body,
        out_type=jax.ShapeDtypeStruct((NC, N_PAD, F), jnp.float32),
        mesh=_get_mesh(),
        compiler_params=pltpu.CompilerParams(needs_layout_passes=False),
        scratch_types=[
            pltpu.VMEM((K,), jnp.int32),
            pltpu.VMEM((K,), jnp.int32),
            pltpu.VMEM((K, F), jnp.float32),
            pltpu.VMEM((K, F), jnp.float32),
            pltpu.SemaphoreType.DMA,
            pltpu.VMEM_SHARED((N_PAD, F), jnp.float32),
        ],
    )(y, g, row, col, zero)


def _dot(a, b):
    return jnp.dot(a, b, preferred_element_type=jnp.float32)


def _residual(x, dW, db, rW, rb):
    h = jnp.maximum(x, 0.0)
    return x + _dot(_dot(h, dW) + db, rW) + rb


def _embed_body(zf_ref, emb_ref, w_ref, b_ref, h_ref, y_ref):
    z = zf_ref[0, 0, :]
    ids = lax.broadcasted_iota(jnp.int32, (256, F), 1).astype(jnp.float32)
    oh = (z[:, None] == ids).astype(jnp.float32)
    h = _dot(oh, emb_ref[...])
    h_ref[...] = h
    y_ref[...] = _dot(jnp.maximum(h, 0.0), w_ref[...]) + b_ref[...]


@jax.jit
def _tc_embed(zf, emb, w, b):
    full = lambda shp: pl.BlockSpec(shp, lambda i: (0, 0))
    return pl.pallas_call(
        _embed_body,
        grid=(NB,),
        in_specs=[
            pl.BlockSpec((1, 1, 256), lambda i: (i, 0, 0)),
            full((F, F)), full((F, F)), full((1, F)),
        ],
        out_specs=[
            pl.BlockSpec((256, F), lambda i: (i, 0)),
            pl.BlockSpec((256, F), lambda i: (i, 0)),
        ],
        out_shape=[
            jax.ShapeDtypeStruct((N_PAD, F), jnp.float32),
            jax.ShapeDtypeStruct((N_PAD, F), jnp.float32),
        ],
    )(zf, emb, w, b)


def _rbf_body(d2_ref, cen_ref, wid_ref, w1_ref, b1_ref, w2_ref, b2_ref,
              g1_ref, g2_ref):
    d2 = d2_ref[0, 0, :]
    ew = jnp.sqrt(d2 + 1e-9)
    x = ew * (1.0 / CUT)
    x3 = x * x * x
    x4 = x3 * x
    x5 = x4 * x
    cutf = jnp.where(x < 1.0, 1.0 - 6.0 * x5 + 15.0 * x4 - 10.0 * x3, 0.0)
    dif = jnp.exp(-ew)[:, None] - cen_ref[...]
    ea = cutf[:, None] * jnp.exp(-wid_ref[...] * dif * dif)
    g1_ref[...] = _dot(ea, w1_ref[...]) + b1_ref[...]
    g2_ref[...] = _dot(ea, w2_ref[...]) + b2_ref[...]


@jax.jit
def _tc_rbf(d2r, cen, wid, w1, b1, w2, b2):
    full = lambda shp: pl.BlockSpec(shp, lambda i: (0, 0))
    return pl.pallas_call(
        _rbf_body,
        grid=(EROWS,),
        in_specs=[
            pl.BlockSpec((1, 1, BE), lambda i: (i, 0, 0)),
            full((1, F)), full((1, F)),
            full((F, F)), full((1, F)), full((F, F)), full((1, F)),
        ],
        out_specs=[
            pl.BlockSpec((BE, F), lambda i: (i, 0)),
            pl.BlockSpec((BE, F), lambda i: (i, 0)),
        ],
        out_shape=[
            jax.ShapeDtypeStruct((E_PAD, F), jnp.float32),
            jax.ShapeDtypeStruct((E_PAD, F), jnp.float32),
        ],
    )(d2r, cen, wid, w1, b1, w2, b2)


def _update(h, ma, mb, ws):
    m = ma + mb
    m = _residual(m, ws["r0dW"], ws["r0db"], ws["r0rW"], ws["r0rb"])
    m = _residual(m, ws["r1dW"], ws["r1db"], ws["r1rW"], ws["r1rb"])
    m = jnp.maximum(m, 0.0)
    x = ws["u"] * h + _dot(m, ws["dW"]) + ws["db"]
    x = _residual(x, ws["a0dW"], ws["a0db"], ws["a0rW"], ws["a0rb"])
    x = _residual(x, ws["a1dW"], ws["a1db"], ws["a1rW"], ws["a1rb"])
    return x


_BLK_KEYS = ("r0dW", "r0db", "r0rW", "r0rb", "r1dW", "r1db", "r1rW", "r1rb",
             "dW", "db", "u",
             "a0dW", "a0db", "a0rW", "a0rb", "a1dW", "a1db", "a1rW", "a1rb")


def _block_body(h_ref, ma_ref, mb_ref, *refs):
    wrefs = refs[:len(_BLK_KEYS)]
    wi_ref, bi_ref, h1_ref, y2_ref = refs[len(_BLK_KEYS):]
    ws = {k: r[...] for k, r in zip(_BLK_KEYS, wrefs)}
    x = _update(h_ref[...], ma_ref[...], mb_ref[...], ws)
    h1_ref[...] = x
    y2_ref[...] = _dot(jnp.maximum(x, 0.0), wi_ref[...]) + bi_ref[...]


@jax.jit
def _tc_block(h, ma, mb, wlist, wi, bi):
    full2 = lambda a: pl.BlockSpec(a.shape, lambda i: (0, 0))
    nspec = pl.BlockSpec((256, F), lambda i: (i, 0))
    return pl.pallas_call(
        _block_body,
        grid=(NB,),
        in_specs=[nspec, nspec, nspec]
                 + [full2(a) for a in wlist] + [full2(wi), full2(bi)],
        out_specs=[nspec, nspec],
        out_shape=[
            jax.ShapeDtypeStruct((N_PAD, F), jnp.float32),
            jax.ShapeDtypeStruct((N_PAD, F), jnp.float32),
        ],
    )(h, ma, mb, *wlist, wi, bi)


def _final_body(h_ref, ma_ref, mb_ref, *refs):
    wrefs = refs[:len(_BLK_KEYS)]
    pw_ref, pb_ref, lw_ref, lb_ref, bf_ref, out_ref = refs[len(_BLK_KEYS):]
    ws = {k: r[...] for k, r in zip(_BLK_KEYS, wrefs)}
    x = _update(h_ref[...], ma_ref[...], mb_ref[...], ws)
    x2 = jnp.maximum(_dot(x, pw_ref[...]) + pb_ref[...], 0.0)
    xx = _dot(x2, lw_ref[...]) + lb_ref[...]
    bfv = bf_ref[0, 0, :]
    gid = lax.broadcasted_iota(jnp.int32, (G, 256), 0).astype(jnp.float32)
    oht = (gid == bfv[None, :]).astype(jnp.float32)
    i = pl.program_id(0)

    @pl.when(i == 0)
    def _init():
        out_ref[...] = jnp.zeros_like(out_ref)

    out_ref[...] += _dot(oht, xx)


@jax.jit
def _tc_final(h, ma, mb, wlist, pw, pb, lw, lb, bf):
    full2 = lambda a: pl.BlockSpec(a.shape, lambda i: (0, 0))
    nspec = pl.BlockSpec((256, F), lambda i: (i, 0))
    return pl.pallas_call(
        _final_body,
        grid=(NB,),
        in_specs=[nspec, nspec, nspec]
                 + [full2(a) for a in wlist]
                 + [full2(pw), full2(pb), full2(lw),
                    pl.BlockSpec((1, 1), lambda i: (0, 0)),
                    pl.BlockSpec((1, 1, 256), lambda i: (i, 0, 0))],
        out_specs=pl.BlockSpec((G, 1), lambda i: (0, 0)),
        out_shape=jax.ShapeDtypeStruct((G, 1), jnp.float32),
    )(h, ma, mb, *wlist, pw, pb, lw, lb, bf)


def _row2(v):
    return v.reshape(1, -1).astype(jnp.float32)


def _pad_cols(a, n):
    return jnp.pad(a, ((0, 0), (0, n - a.shape[1])))


def _pad_rows(a, n):
    return jnp.pad(a, ((0, n - a.shape[0]), (0, 0)))


def _block_wlist(p):
    r0, r1 = p['res']
    a0, a1 = p['ares']
    vals = (r0['dW'], _row2(r0['db']), r0['rW'], _row2(r0['rb']),
            r1['dW'], _row2(r1['db']), r1['rW'], _row2(r1['rb']),
            p['dense_W'], _row2(p['dense_b']), _row2(p['u']),
            a0['dW'], _row2(a0['db']), a0['rW'], _row2(a0['rb']),
            a1['dW'], _row2(a1['db']), a1['rW'], _row2(a1['rb']))
    return list(vals)


def kernel(z, pos, batch, edge_index, params):
    row = edge_index[0].astype(jnp.int32)
    col = edge_index[1].astype(jnp.int32)
    row_p = jnp.pad(row, (0, E_PAD - E))
    col_p = jnp.pad(col, (0, E_PAD - E), constant_values=N)
    posf = pos.astype(jnp.float32)
    px = jnp.pad(posf[:, 0], (0, N_PAD - N))
    py = jnp.pad(posf[:, 1], (0, N_PAD - N))
    pz = jnp.pad(posf[:, 2], (0, N_PAD - N))
    zf = jnp.pad(z.astype(jnp.float32), (0, N_PAD - N)).reshape(NB, 1, 256)
    bf = jnp.pad(batch.astype(jnp.float32), (0, N_PAD - N),
                 constant_values=float(2 * G)).reshape(NB, 1, 256)

    p = params
    emb = _pad_rows(p['emb'], F)
    b0, b1 = p['blocks']
    cen = _pad_cols(_row2(p['centers']), F)
    wid = _pad_cols(_row2(p['widths']), F)
    w1 = _pad_rows(b0['d2f_W'], F)
    w2 = _pad_rows(b1['d2f_W'], F)
    gb1 = _row2(b0['d2f_b'])
    gb2 = _row2(b1['d2f_b'])
    pw = _pad_cols(p['post_W'], F)
    pb = _pad_cols(_row2(p['post_b']), F)
    lw = _pad_rows(p['last_W'], F)
    lb = _row2(p['last_b'])

    zero = jnp.zeros((N_PAD, F), jnp.float32)

    d2 = _sc_d2(row_p, col_p, px, py, pz)
    h, y1 = _tc_embed(zf, emb, b0['di_W'], _row2(b0['di_b']))
    g1, g2 = _tc_rbf(d2.reshape(EROWS, 1, BE), cen, wid, w1, gb1, w2, gb2)
    mp1 = _sc_msg(y1, g1, row_p, col_p, zero)
    h1, y2 = _tc_block(h, mp1[0], mp1[1], _block_wlist(b0),
                       b1['di_W'], _row2(b1['di_b']))
    mp2 = _sc_msg(y2, g2, row_p, col_p, zero)
    out = _tc_final(h1, mp2[0], mp2[1], _block_wlist(b1), pw, pb, lw, lb, bf)
    return out


# speedup vs baseline: 3.1124x; 1.1488x over previous
import functools

import jax
import jax.numpy as jnp
from jax import lax
from jax.experimental import pallas as pl
from jax.experimental.pallas import tpu as pltpu
from jax.experimental.pallas import tpu_sc as plsc

N = 10000
E = 320000
G = 256
F = 128
D = 100
CUT = 12.0

NC = 2
NS = 16
L = 16
NW = NC * NS

K = 64
E_PAD = 327680
EPW = E_PAD // NW
CHUNKS = EPW // K
N_PAD = 10240
NSLICE = N_PAD // NS
BE = 512
EROWS = E_PAD // BE
NB = N_PAD // 256

@functools.cache
def _get_mesh():
    return plsc.VectorSubcoreMesh(core_axis_name="c", subcore_axis_name="s",
                                  num_cores=NC, num_subcores=NS)


def _sc_d2_body(row_hbm, col_hbm, px_hbm, py_hbm, pz_hbm, d2_hbm,
                row_v, col_v, px_v, py_v, pz_v, d2_v):
    cid = lax.axis_index("c")
    sid = lax.axis_index("s")
    wid = sid * NC + cid
    base = wid * EPW
    pltpu.sync_copy(row_hbm.at[pl.ds(base, EPW)], row_v)
    pltpu.sync_copy(col_hbm.at[pl.ds(base, EPW)], col_v)
    pltpu.sync_copy(px_hbm, px_v)
    pltpu.sync_copy(py_hbm, py_v)
    pltpu.sync_copy(pz_hbm, pz_v)

    def step(i, carry):
        sl = pl.ds(i * L, L)
        r = row_v[sl]
        c = col_v[sl]
        dx = plsc.load_gather(px_v, [r]) - plsc.load_gather(px_v, [c])
        dy = plsc.load_gather(py_v, [r]) - plsc.load_gather(py_v, [c])
        dz = plsc.load_gather(pz_v, [r]) - plsc.load_gather(pz_v, [c])
        d2_v[sl] = dx * dx + dy * dy + dz * dz
        return carry

    lax.fori_loop(0, EPW // L, step, 0)
    pltpu.sync_copy(d2_v, d2_hbm.at[pl.ds(base, EPW)])


@jax.jit
def _sc_d2(row, col, px, py, pz):
    return pl.kernel(
        _sc_d2_body,
        out_type=jax.ShapeDtypeStruct((E_PAD,), jnp.float32),
        mesh=_get_mesh(),
        compiler_params=pltpu.CompilerParams(needs_layout_passes=False),
        scratch_types=[
            pltpu.VMEM((EPW,), jnp.int32),
            pltpu.VMEM((EPW,), jnp.int32),
            pltpu.VMEM((N_PAD,), jnp.float32),
            pltpu.VMEM((N_PAD,), jnp.float32),
            pltpu.VMEM((N_PAD,), jnp.float32),
            pltpu.VMEM((EPW,), jnp.float32),
        ],
    )(row, col, px, py, pz)


def _sc_msg_body(y_hbm, g_hbm, row_hbm, col_hbm, zero_hbm, out_hbm,
                 row_v, col0, col1, y_bufs, g_bufs, semy, semg, semc, m_sh):
    cid = lax.axis_index("c")
    sid = lax.axis_index("s")
    wid = sid * NC + cid
    base = wid * EPW
    zsl = pl.ds(sid * NSLICE, NSLICE)
    cols = (col0, col1)
    pltpu.sync_copy(zero_hbm.at[zsl], m_sh.at[zsl])
    pltpu.sync_copy(row_hbm.at[pl.ds(base, EPW)], row_v)
    plsc.subcore_barrier()

    def issue(k, p):
        pltpu.async_copy(g_hbm.at[pl.ds(base + k * K, K)], g_bufs.at[p],
                         semg.at[p])
        pltpu.async_copy(y_hbm.at[row_v.at[pl.ds(k * K, K)]], y_bufs.at[p],
                         semy.at[p])
        pltpu.async_copy(col_hbm.at[pl.ds(base + k * K, K)], cols[p],
                         semc.at[p])

    def consume(k, p):
        pltpu.make_async_copy(g_hbm.at[pl.ds(base + k * K, K)], g_bufs.at[p],
                              semg.at[p]).wait()
        pltpu.make_async_copy(y_hbm.at[row_v.at[pl.ds(k * K, K)]],
                              y_bufs.at[p], semy.at[p]).wait()
        pltpu.make_async_copy(col_hbm.at[pl.ds(base + k * K, K)], cols[p],
                              semc.at[p]).wait()

        def mul_row(r4, c2):
            for rr in range(4):
                for kk in range(F // L):
                    sl = pl.ds(kk * L, L)
                    g_bufs[p, r4 * 4 + rr, sl] = (g_bufs[p, r4 * 4 + rr, sl]
                                                  * y_bufs[p, r4 * 4 + rr, sl])
            return c2

        lax.fori_loop(0, K // 4, mul_row, 0)
        pltpu.sync_copy(g_bufs.at[p], m_sh.at[cols[p]], add=True)

    issue(0, 0)

    def pair(kk, carry):
        for p in range(2):
            k = kk * 2 + p

            @pl.when(k + 1 < CHUNKS)
            def _pre():
                issue(k + 1, 1 - p)

            consume(k, p)
        return carry

    lax.fori_loop(0, CHUNKS // 2, pair, 0)
    plsc.subcore_barrier()
    pltpu.sync_copy(m_sh.at[zsl], out_hbm.at[cid, zsl])


@jax.jit
def _sc_msg(y, g, row, col, zero):
    return pl.kernel(
        _sc_msg_body,
        out_type=jax.ShapeDtypeStruct((NC, N_PAD, F), jnp.float32),
        mesh=_get_mesh(),
        compiler_params=pltpu.CompilerParams(needs_layout_passes=False),
        scratch_types=[
            pltpu.VMEM((EPW,), jnp.int32),
            pltpu.VMEM((K,), jnp.int32),
            pltpu.VMEM((K,), jnp.int32),
            pltpu.VMEM((2, K, F), jnp.float32),
            pltpu.VMEM((2, K, F), jnp.float32),
            pltpu.SemaphoreType.DMA((2,)),
            pltpu.SemaphoreType.DMA((2,)),
            pltpu.SemaphoreType.DMA((2,)),
            pltpu.VMEM_SHARED((N_PAD, F), jnp.float32),
        ],
    )(y, g, row, col, zero)


def _dot(a, b):
    return jnp.dot(a, b, preferred_element_type=jnp.float32)


def _residual(x, dW, db, rW, rb):
    h = jnp.maximum(x, 0.0)
    return x + _dot(_dot(h, dW) + db, rW) + rb


def _embed_body(zf_ref, emb_ref, w_ref, b_ref, h_ref, y_ref):
    z = zf_ref[0, 0, :]
    ids = lax.broadcasted_iota(jnp.int32, (256, F), 1).astype(jnp.float32)
    oh = (z[:, None] == ids).astype(jnp.float32)
    h = _dot(oh, emb_ref[...])
    h_ref[...] = h
    y_ref[...] = _dot(jnp.maximum(h, 0.0), w_ref[...]) + b_ref[...]


@jax.jit
def _tc_embed(zf, emb, w, b):
    full = lambda shp: pl.BlockSpec(shp, lambda i: (0, 0))
    return pl.pallas_call(
        _embed_body,
        grid=(NB,),
        in_specs=[
            pl.BlockSpec((1, 1, 256), lambda i: (i, 0, 0)),
            full((F, F)), full((F, F)), full((1, F)),
        ],
        out_specs=[
            pl.BlockSpec((256, F), lambda i: (i, 0)),
            pl.BlockSpec((256, F), lambda i: (i, 0)),
        ],
        out_shape=[
            jax.ShapeDtypeStruct((N_PAD, F), jnp.float32),
            jax.ShapeDtypeStruct((N_PAD, F), jnp.float32),
        ],
    )(zf, emb, w, b)


def _rbf_body(d2_ref, cen_ref, wid_ref, w1_ref, b1_ref, w2_ref, b2_ref,
              g1_ref, g2_ref):
    d2 = d2_ref[0, 0, :]
    ew = jnp.sqrt(d2 + 1e-9)
    x = ew * (1.0 / CUT)
    x3 = x * x * x
    x4 = x3 * x
    x5 = x4 * x
    cutf = jnp.where(x < 1.0, 1.0 - 6.0 * x5 + 15.0 * x4 - 10.0 * x3, 0.0)
    dif = jnp.exp(-ew)[:, None] - cen_ref[...]
    ea = cutf[:, None] * jnp.exp(-wid_ref[...] * dif * dif)
    g1_ref[...] = _dot(ea, w1_ref[...]) + b1_ref[...]
    g2_ref[...] = _dot(ea, w2_ref[...]) + b2_ref[...]


@jax.jit
def _tc_rbf(d2r, cen, wid, w1, b1, w2, b2):
    full = lambda shp: pl.BlockSpec(shp, lambda i: (0, 0))
    return pl.pallas_call(
        _rbf_body,
        grid=(EROWS,),
        in_specs=[
            pl.BlockSpec((1, 1, BE), lambda i: (i, 0, 0)),
            full((1, F)), full((1, F)),
            full((F, F)), full((1, F)), full((F, F)), full((1, F)),
        ],
        out_specs=[
            pl.BlockSpec((BE, F), lambda i: (i, 0)),
            pl.BlockSpec((BE, F), lambda i: (i, 0)),
        ],
        out_shape=[
            jax.ShapeDtypeStruct((E_PAD, F), jnp.float32),
            jax.ShapeDtypeStruct((E_PAD, F), jnp.float32),
        ],
    )(d2r, cen, wid, w1, b1, w2, b2)


def _update(h, ma, mb, ws):
    m = ma + mb
    m = _residual(m, ws["r0dW"], ws["r0db"], ws["r0rW"], ws["r0rb"])
    m = _residual(m, ws["r1dW"], ws["r1db"], ws["r1rW"], ws["r1rb"])
    m = jnp.maximum(m, 0.0)
    x = ws["u"] * h + _dot(m, ws["dW"]) + ws["db"]
    x = _residual(x, ws["a0dW"], ws["a0db"], ws["a0rW"], ws["a0rb"])
    x = _residual(x, ws["a1dW"], ws["a1db"], ws["a1rW"], ws["a1rb"])
    return x


_BLK_KEYS = ("r0dW", "r0db", "r0rW", "r0rb", "r1dW", "r1db", "r1rW", "r1rb",
             "dW", "db", "u",
             "a0dW", "a0db", "a0rW", "a0rb", "a1dW", "a1db", "a1rW", "a1rb")


def _block_body(h_ref, ma_ref, mb_ref, *refs):
    wrefs = refs[:len(_BLK_KEYS)]
    wi_ref, bi_ref, h1_ref, y2_ref = refs[len(_BLK_KEYS):]
    ws = {k: r[...] for k, r in zip(_BLK_KEYS, wrefs)}
    x = _update(h_ref[...], ma_ref[...], mb_ref[...], ws)
    h1_ref[...] = x
    y2_ref[...] = _dot(jnp.maximum(x, 0.0), wi_ref[...]) + bi_ref[...]


@jax.jit
def _tc_block(h, ma, mb, wlist, wi, bi):
    full2 = lambda a: pl.BlockSpec(a.shape, lambda i: (0, 0))
    nspec = pl.BlockSpec((256, F), lambda i: (i, 0))
    return pl.pallas_call(
        _block_body,
        grid=(NB,),
        in_specs=[nspec, nspec, nspec]
                 + [full2(a) for a in wlist] + [full2(wi), full2(bi)],
        out_specs=[nspec, nspec],
        out_shape=[
            jax.ShapeDtypeStruct((N_PAD, F), jnp.float32),
            jax.ShapeDtypeStruct((N_PAD, F), jnp.float32),
        ],
    )(h, ma, mb, *wlist, wi, bi)


def _final_body(h_ref, ma_ref, mb_ref, *refs):
    wrefs = refs[:len(_BLK_KEYS)]
    pw_ref, pb_ref, lw_ref, lb_ref, bf_ref, out_ref = refs[len(_BLK_KEYS):]
    ws = {k: r[...] for k, r in zip(_BLK_KEYS, wrefs)}
    x = _update(h_ref[...], ma_ref[...], mb_ref[...], ws)
    x2 = jnp.maximum(_dot(x, pw_ref[...]) + pb_ref[...], 0.0)
    xx = _dot(x2, lw_ref[...]) + lb_ref[...]
    bfv = bf_ref[0, 0, :]
    gid = lax.broadcasted_iota(jnp.int32, (G, 256), 0).astype(jnp.float32)
    oht = (gid == bfv[None, :]).astype(jnp.float32)
    i = pl.program_id(0)

    @pl.when(i == 0)
    def _init():
        out_ref[...] = jnp.zeros_like(out_ref)

    out_ref[...] += _dot(oht, xx)


@jax.jit
def _tc_final(h, ma, mb, wlist, pw, pb, lw, lb, bf):
    full2 = lambda a: pl.BlockSpec(a.shape, lambda i: (0, 0))
    nspec = pl.BlockSpec((256, F), lambda i: (i, 0))
    return pl.pallas_call(
        _final_body,
        grid=(NB,),
        in_specs=[nspec, nspec, nspec]
                 + [full2(a) for a in wlist]
                 + [full2(pw), full2(pb), full2(lw),
                    pl.BlockSpec((1, 1), lambda i: (0, 0)),
                    pl.BlockSpec((1, 1, 256), lambda i: (i, 0, 0))],
        out_specs=pl.BlockSpec((G, 1), lambda i: (0, 0)),
        out_shape=jax.ShapeDtypeStruct((G, 1), jnp.float32),
    )(h, ma, mb, *wlist, pw, pb, lw, lb, bf)


def _row2(v):
    return v.reshape(1, -1).astype(jnp.float32)


def _pad_cols(a, n):
    return jnp.pad(a, ((0, 0), (0, n - a.shape[1])))


def _pad_rows(a, n):
    return jnp.pad(a, ((0, n - a.shape[0]), (0, 0)))


def _block_wlist(p):
    r0, r1 = p['res']
    a0, a1 = p['ares']
    vals = (r0['dW'], _row2(r0['db']), r0['rW'], _row2(r0['rb']),
            r1['dW'], _row2(r1['db']), r1['rW'], _row2(r1['rb']),
            p['dense_W'], _row2(p['dense_b']), _row2(p['u']),
            a0['dW'], _row2(a0['db']), a0['rW'], _row2(a0['rb']),
            a1['dW'], _row2(a1['db']), a1['rW'], _row2(a1['rb']))
    return list(vals)


def kernel(z, pos, batch, edge_index, params):
    row = edge_index[0].astype(jnp.int32)
    col = edge_index[1].astype(jnp.int32)
    row_p = jnp.pad(row, (0, E_PAD - E))
    col_p = jnp.pad(col, (0, E_PAD - E), constant_values=N)
    posf = pos.astype(jnp.float32)
    px = jnp.pad(posf[:, 0], (0, N_PAD - N))
    py = jnp.pad(posf[:, 1], (0, N_PAD - N))
    pz = jnp.pad(posf[:, 2], (0, N_PAD - N))
    zf = jnp.pad(z.astype(jnp.float32), (0, N_PAD - N)).reshape(NB, 1, 256)
    bf = jnp.pad(batch.astype(jnp.float32), (0, N_PAD - N),
                 constant_values=float(2 * G)).reshape(NB, 1, 256)

    p = params
    emb = _pad_rows(p['emb'], F)
    b0, b1 = p['blocks']
    cen = _pad_cols(_row2(p['centers']), F)
    wid = _pad_cols(_row2(p['widths']), F)
    w1 = _pad_rows(b0['d2f_W'], F)
    w2 = _pad_rows(b1['d2f_W'], F)
    gb1 = _row2(b0['d2f_b'])
    gb2 = _row2(b1['d2f_b'])
    pw = _pad_cols(p['post_W'], F)
    pb = _pad_cols(_row2(p['post_b']), F)
    lw = _pad_rows(p['last_W'], F)
    lb = _row2(p['last_b'])

    zero = jnp.zeros((N_PAD, F), jnp.float32)

    d2 = _sc_d2(row_p, col_p, px, py, pz)
    h, y1 = _tc_embed(zf, emb, b0['di_W'], _row2(b0['di_b']))
    g1, g2 = _tc_rbf(d2.reshape(EROWS, 1, BE), cen, wid, w1, gb1, w2, gb2)
    mp1 = _sc_msg(y1, g1, row_p, col_p, zero)
    h1, y2 = _tc_block(h, mp1[0], mp1[1], _block_wlist(b0),
                       b1['di_W'], _row2(b1['di_b']))
    mp2 = _sc_msg(y2, g2, row_p, col_p, zero)
    out = _tc_final(h1, mp2[0], mp2[1], _block_wlist(b1), pw, pb, lw, lb, bf)
    return out


# speedup vs baseline: 4.2679x; 1.3712x over previous
import functools

import jax
import jax.numpy as jnp
from jax import lax
from jax.experimental import pallas as pl
from jax.experimental.pallas import tpu as pltpu
from jax.experimental.pallas import tpu_sc as plsc

N = 10000
E = 320000
G = 256
F = 128
D = 100
CUT = 12.0

NC = 2
NS = 16
L = 16
NW = NC * NS

K = 48
E_PAD = 322560
EPW = E_PAD // NW
CHUNKS = EPW // K
N_PAD = 10240
NSLICE = N_PAD // NS
BE = 512
EROWS = E_PAD // BE
NB = N_PAD // 256

@functools.cache
def _get_mesh():
    return plsc.VectorSubcoreMesh(core_axis_name="c", subcore_axis_name="s",
                                  num_cores=NC, num_subcores=NS)


def _sc_d2_body(row_hbm, col_hbm, px_hbm, py_hbm, pz_hbm, d2_hbm,
                row_v, col_v, px_v, py_v, pz_v, d2_v):
    cid = lax.axis_index("c")
    sid = lax.axis_index("s")
    wid = sid * NC + cid
    base = wid * EPW
    pltpu.sync_copy(row_hbm.at[pl.ds(base, EPW)], row_v)
    pltpu.sync_copy(col_hbm.at[pl.ds(base, EPW)], col_v)
    pltpu.sync_copy(px_hbm, px_v)
    pltpu.sync_copy(py_hbm, py_v)
    pltpu.sync_copy(pz_hbm, pz_v)

    def step(i, carry):
        sl = pl.ds(i * L, L)
        r = row_v[sl]
        c = col_v[sl]
        dx = plsc.load_gather(px_v, [r]) - plsc.load_gather(px_v, [c])
        dy = plsc.load_gather(py_v, [r]) - plsc.load_gather(py_v, [c])
        dz = plsc.load_gather(pz_v, [r]) - plsc.load_gather(pz_v, [c])
        d2_v[sl] = dx * dx + dy * dy + dz * dz
        return carry

    lax.fori_loop(0, EPW // L, step, 0)
    pltpu.sync_copy(d2_v, d2_hbm.at[pl.ds(base, EPW)])


@jax.jit
def _sc_d2(row, col, px, py, pz):
    return pl.kernel(
        _sc_d2_body,
        out_type=jax.ShapeDtypeStruct((E_PAD,), jnp.float32),
        mesh=_get_mesh(),
        compiler_params=pltpu.CompilerParams(needs_layout_passes=False),
        scratch_types=[
            pltpu.VMEM((EPW,), jnp.int32),
            pltpu.VMEM((EPW,), jnp.int32),
            pltpu.VMEM((N_PAD,), jnp.float32),
            pltpu.VMEM((N_PAD,), jnp.float32),
            pltpu.VMEM((N_PAD,), jnp.float32),
            pltpu.VMEM((EPW,), jnp.float32),
        ],
    )(row, col, px, py, pz)


def _sc_msg_body(y_hbm, g_hbm, row_hbm, col_hbm, zero_hbm, out_hbm,
                 row_v, col0, col1, col2, y_bufs, g_bufs,
                 semy, semg, semc, sems, m_sh):
    cid = lax.axis_index("c")
    sid = lax.axis_index("s")
    wid = sid * NC + cid
    base = wid * EPW
    zsl = pl.ds(sid * NSLICE, NSLICE)
    cols = (col0, col1, col2)
    pltpu.sync_copy(zero_hbm.at[zsl], m_sh.at[zsl])
    pltpu.sync_copy(row_hbm.at[pl.ds(base, EPW)], row_v)
    plsc.subcore_barrier()

    def issue(k, p):
        pltpu.async_copy(g_hbm.at[pl.ds(base + k * K, K)], g_bufs.at[p],
                         semg.at[p])
        pltpu.async_copy(y_hbm.at[row_v.at[pl.ds(k * K, K)]], y_bufs.at[p],
                         semy.at[p])
        pltpu.async_copy(col_hbm.at[pl.ds(base + k * K, K)], cols[p],
                         semc.at[p])

    def scat_wait(p):
        pltpu.make_async_copy(g_bufs.at[p], m_sh.at[cols[p]],
                              sems.at[p]).wait()

    def consume(k, p):
        pltpu.make_async_copy(g_hbm.at[pl.ds(base + k * K, K)], g_bufs.at[p],
                              semg.at[p]).wait()
        pltpu.make_async_copy(y_hbm.at[row_v.at[pl.ds(k * K, K)]],
                              y_bufs.at[p], semy.at[p]).wait()
        pltpu.make_async_copy(col_hbm.at[pl.ds(base + k * K, K)], cols[p],
                              semc.at[p]).wait()

        def mul_row(r4, c2):
            for rr in range(4):
                for kk in range(F // L):
                    sl = pl.ds(kk * L, L)
                    g_bufs[p, r4 * 4 + rr, sl] = (g_bufs[p, r4 * 4 + rr, sl]
                                                  * y_bufs[p, r4 * 4 + rr, sl])
            return c2

        lax.fori_loop(0, K // 4, mul_row, 0)
        pltpu.async_copy(g_bufs.at[p], m_sh.at[cols[p]], sems.at[p], add=True)

    issue(0, 0)
    issue(1, 1)

    def trip(kk, carry):
        for p in range(3):
            k = kk * 3 + p

            @pl.when(k + 2 < CHUNKS)
            def _pre():
                q = (p + 2) % 3

                @pl.when(k >= 1)
                def _drain():
                    scat_wait(q)

                issue(k + 2, q)

            consume(k, p)
        return carry

    lax.fori_loop(0, CHUNKS // 3, trip, 0)
    for p in range(3):
        scat_wait(p)
    plsc.subcore_barrier()
    pltpu.sync_copy(m_sh.at[zsl], out_hbm.at[cid, zsl])


@jax.jit
def _sc_msg(y, g, row, col, zero):
    return pl.kernel(
        _sc_msg_body,
        out_type=jax.ShapeDtypeStruct((NC, N_PAD, F), jnp.float32),
        mesh=_get_mesh(),
        compiler_params=pltpu.CompilerParams(needs_layout_passes=False),
        scratch_types=[
            pltpu.VMEM((EPW,), jnp.int32),
            pltpu.VMEM((K,), jnp.int32),
            pltpu.VMEM((K,), jnp.int32),
            pltpu.VMEM((K,), jnp.int32),
            pltpu.VMEM((3, K, F), jnp.float32),
            pltpu.VMEM((3, K, F), jnp.float32),
            pltpu.SemaphoreType.DMA((3,)),
            pltpu.SemaphoreType.DMA((3,)),
            pltpu.SemaphoreType.DMA((3,)),
            pltpu.SemaphoreType.DMA((3,)),
            pltpu.VMEM_SHARED((N_PAD, F), jnp.float32),
        ],
    )(y, g, row, col, zero)


def _dot(a, b):
    return jnp.dot(a, b, preferred_element_type=jnp.float32)


def _residual(x, dW, db, rW, rb):
    h = jnp.maximum(x, 0.0)
    return x + _dot(_dot(h, dW) + db, rW) + rb


def _embed_body(zf_ref, emb_ref, w_ref, b_ref, h_ref, y_ref):
    z = zf_ref[0, 0, :]
    ids = lax.broadcasted_iota(jnp.int32, (256, F), 1).astype(jnp.float32)
    oh = (z[:, None] == ids).astype(jnp.float32)
    h = _dot(oh, emb_ref[...])
    h_ref[...] = h
    y_ref[...] = _dot(jnp.maximum(h, 0.0), w_ref[...]) + b_ref[...]


@jax.jit
def _tc_embed(zf, emb, w, b):
    full = lambda shp: pl.BlockSpec(shp, lambda i: (0, 0))
    return pl.pallas_call(
        _embed_body,
        grid=(NB,),
        in_specs=[
            pl.BlockSpec((1, 1, 256), lambda i: (i, 0, 0)),
            full((F, F)), full((F, F)), full((1, F)),
        ],
        out_specs=[
            pl.BlockSpec((256, F), lambda i: (i, 0)),
            pl.BlockSpec((256, F), lambda i: (i, 0)),
        ],
        out_shape=[
            jax.ShapeDtypeStruct((N_PAD, F), jnp.float32),
            jax.ShapeDtypeStruct((N_PAD, F), jnp.float32),
        ],
    )(zf, emb, w, b)


def _rbf_body(d2_ref, cen_ref, wid_ref, w1_ref, b1_ref, w2_ref, b2_ref,
              g1_ref, g2_ref):
    d2 = d2_ref[0, 0, :]
    ew = jnp.sqrt(d2 + 1e-9)
    x = ew * (1.0 / CUT)
    x3 = x * x * x
    x4 = x3 * x
    x5 = x4 * x
    cutf = jnp.where(x < 1.0, 1.0 - 6.0 * x5 + 15.0 * x4 - 10.0 * x3, 0.0)
    dif = jnp.exp(-ew)[:, None] - cen_ref[...]
    ea = cutf[:, None] * jnp.exp(-wid_ref[...] * dif * dif)
    eab = ea.astype(jnp.bfloat16)
    g1_ref[...] = _dot(eab, w1_ref[...]) + b1_ref[...]
    g2_ref[...] = _dot(eab, w2_ref[...]) + b2_ref[...]


@jax.jit
def _tc_rbf(d2r, cen, wid, w1, b1, w2, b2):
    full = lambda shp: pl.BlockSpec(shp, lambda i: (0, 0))
    return pl.pallas_call(
        _rbf_body,
        grid=(EROWS,),
        in_specs=[
            pl.BlockSpec((1, 1, BE), lambda i: (i, 0, 0)),
            full((1, F)), full((1, F)),
            full((F, F)), full((1, F)), full((F, F)), full((1, F)),
        ],
        out_specs=[
            pl.BlockSpec((BE, F), lambda i: (i, 0)),
            pl.BlockSpec((BE, F), lambda i: (i, 0)),
        ],
        out_shape=[
            jax.ShapeDtypeStruct((E_PAD, F), jnp.float32),
            jax.ShapeDtypeStruct((E_PAD, F), jnp.float32),
        ],
    )(d2r, cen, wid, w1, b1, w2, b2)


def _update(h, ma, mb, ws):
    m = ma + mb
    m = _residual(m, ws["r0dW"], ws["r0db"], ws["r0rW"], ws["r0rb"])
    m = _residual(m, ws["r1dW"], ws["r1db"], ws["r1rW"], ws["r1rb"])
    m = jnp.maximum(m, 0.0)
    x = ws["u"] * h + _dot(m, ws["dW"]) + ws["db"]
    x = _residual(x, ws["a0dW"], ws["a0db"], ws["a0rW"], ws["a0rb"])
    x = _residual(x, ws["a1dW"], ws["a1db"], ws["a1rW"], ws["a1rb"])
    return x


_BLK_KEYS = ("r0dW", "r0db", "r0rW", "r0rb", "r1dW", "r1db", "r1rW", "r1rb",
             "dW", "db", "u",
             "a0dW", "a0db", "a0rW", "a0rb", "a1dW", "a1db", "a1rW", "a1rb")


def _block_body(h_ref, ma_ref, mb_ref, *refs):
    wrefs = refs[:len(_BLK_KEYS)]
    wi_ref, bi_ref, h1_ref, y2_ref = refs[len(_BLK_KEYS):]
    ws = {k: r[...] for k, r in zip(_BLK_KEYS, wrefs)}
    x = _update(h_ref[...], ma_ref[...], mb_ref[...], ws)
    h1_ref[...] = x
    y2_ref[...] = _dot(jnp.maximum(x, 0.0), wi_ref[...]) + bi_ref[...]


@jax.jit
def _tc_block(h, ma, mb, wlist, wi, bi):
    full2 = lambda a: pl.BlockSpec(a.shape, lambda i: (0, 0))
    nspec = pl.BlockSpec((256, F), lambda i: (i, 0))
    return pl.pallas_call(
        _block_body,
        grid=(NB,),
        in_specs=[nspec, nspec, nspec]
                 + [full2(a) for a in wlist] + [full2(wi), full2(bi)],
        out_specs=[nspec, nspec],
        out_shape=[
            jax.ShapeDtypeStruct((N_PAD, F), jnp.float32),
            jax.ShapeDtypeStruct((N_PAD, F), jnp.float32),
        ],
    )(h, ma, mb, *wlist, wi, bi)


def _final_body(h_ref, ma_ref, mb_ref, *refs):
    wrefs = refs[:len(_BLK_KEYS)]
    pw_ref, pb_ref, lw_ref, lb_ref, bf_ref, out_ref = refs[len(_BLK_KEYS):]
    ws = {k: r[...] for k, r in zip(_BLK_KEYS, wrefs)}
    x = _update(h_ref[...], ma_ref[...], mb_ref[...], ws)
    x2 = jnp.maximum(_dot(x, pw_ref[...]) + pb_ref[...], 0.0)
    xx = _dot(x2, lw_ref[...]) + lb_ref[...]
    bfv = bf_ref[0, 0, :]
    gid = lax.broadcasted_iota(jnp.int32, (G, 256), 0).astype(jnp.float32)
    oht = (gid == bfv[None, :]).astype(jnp.float32)
    i = pl.program_id(0)

    @pl.when(i == 0)
    def _init():
        out_ref[...] = jnp.zeros_like(out_ref)

    out_ref[...] += _dot(oht, xx)


@jax.jit
def _tc_final(h, ma, mb, wlist, pw, pb, lw, lb, bf):
    full2 = lambda a: pl.BlockSpec(a.shape, lambda i: (0, 0))
    nspec = pl.BlockSpec((256, F), lambda i: (i, 0))
    return pl.pallas_call(
        _final_body,
        grid=(NB,),
        in_specs=[nspec, nspec, nspec]
                 + [full2(a) for a in wlist]
                 + [full2(pw), full2(pb), full2(lw),
                    pl.BlockSpec((1, 1), lambda i: (0, 0)),
                    pl.BlockSpec((1, 1, 256), lambda i: (i, 0, 0))],
        out_specs=pl.BlockSpec((G, 1), lambda i: (0, 0)),
        out_shape=jax.ShapeDtypeStruct((G, 1), jnp.float32),
    )(h, ma, mb, *wlist, pw, pb, lw, lb, bf)


def _row2(v):
    return v.reshape(1, -1).astype(jnp.float32)


def _pad_cols(a, n):
    return jnp.pad(a, ((0, 0), (0, n - a.shape[1])))


def _pad_rows(a, n):
    return jnp.pad(a, ((0, n - a.shape[0]), (0, 0)))


def _block_wlist(p):
    r0, r1 = p['res']
    a0, a1 = p['ares']
    vals = (r0['dW'], _row2(r0['db']), r0['rW'], _row2(r0['rb']),
            r1['dW'], _row2(r1['db']), r1['rW'], _row2(r1['rb']),
            p['dense_W'], _row2(p['dense_b']), _row2(p['u']),
            a0['dW'], _row2(a0['db']), a0['rW'], _row2(a0['rb']),
            a1['dW'], _row2(a1['db']), a1['rW'], _row2(a1['rb']))
    return list(vals)


def kernel(z, pos, batch, edge_index, params):
    row = edge_index[0].astype(jnp.int32)
    col = edge_index[1].astype(jnp.int32)
    row_p = jnp.pad(row, (0, E_PAD - E))
    col_p = jnp.pad(col, (0, E_PAD - E), constant_values=N)
    posf = pos.astype(jnp.float32)
    px = jnp.pad(posf[:, 0], (0, N_PAD - N))
    py = jnp.pad(posf[:, 1], (0, N_PAD - N))
    pz = jnp.pad(posf[:, 2], (0, N_PAD - N))
    zf = jnp.pad(z.astype(jnp.float32), (0, N_PAD - N)).reshape(NB, 1, 256)
    bf = jnp.pad(batch.astype(jnp.float32), (0, N_PAD - N),
                 constant_values=float(2 * G)).reshape(NB, 1, 256)

    p = params
    emb = _pad_rows(p['emb'], F)
    b0, b1 = p['blocks']
    cen = _pad_cols(_row2(p['centers']), F)
    wid = _pad_cols(_row2(p['widths']), F)
    w1 = _pad_rows(b0['d2f_W'], F).astype(jnp.bfloat16)
    w2 = _pad_rows(b1['d2f_W'], F).astype(jnp.bfloat16)
    gb1 = _row2(b0['d2f_b'])
    gb2 = _row2(b1['d2f_b'])
    pw = _pad_cols(p['post_W'], F)
    pb = _pad_cols(_row2(p['post_b']), F)
    lw = _pad_rows(p['last_W'], F)
    lb = _row2(p['last_b'])

    zero = jnp.zeros((N_PAD, F), jnp.float32)

    d2 = _sc_d2(row_p, col_p, px, py, pz)
    h, y1 = _tc_embed(zf, emb, b0['di_W'], _row2(b0['di_b']))
    g1, g2 = _tc_rbf(d2.reshape(EROWS, 1, BE), cen, wid, w1, gb1, w2, gb2)
    mp1 = _sc_msg(y1, g1, row_p, col_p, zero)
    h1, y2 = _tc_block(h, mp1[0], mp1[1], _block_wlist(b0),
                       b1['di_W'], _row2(b1['di_b']))
    mp2 = _sc_msg(y2, g2, row_p, col_p, zero)
    out = _tc_final(h1, mp2[0], mp2[1], _block_wlist(b1), pw, pb, lw, lb, bf)
    return out
